# baseline (speedup 1.0000x reference)
import functools

import jax
import jax.numpy as jnp
from jax import lax
from jax.experimental import pallas as pl
from jax.experimental.pallas import tpu as pltpu
from jax.experimental.pallas import tpu_sc as plsc

_EPS = 1e-12
_L = 16
_NC = 2
_NS = 16
_NW = _NC * _NS
_C = 8


def _rsqrt_newton(x):
    xb = lax.bitcast_convert_type(x, jnp.int32)
    y = lax.bitcast_convert_type(
        jnp.int32(0x5F3759DF) - lax.shift_right_arithmetic(xb, 1), jnp.float32
    )
    for _ in range(4):
        y = y * (1.5 - 0.5 * x * y * y)
    return y


def _lane_sum(v):
    ids = lax.iota(jnp.int32, _L)
    dnums = lax.GatherDimensionNumbers(
        offset_dims=(), collapsed_slice_dims=(0,), start_index_map=(0,)
    )
    for k in (1, 2, 4, 8):
        idx = lax.bitwise_xor(ids, jnp.int32(k))
        v = v + lax.gather(
            v,
            idx[:, None],
            dnums,
            slice_sizes=(1,),
            mode=lax.GatherScatterMode.PROMISE_IN_BOUNDS,
        )
    return v


def _sc_body(in_hbm, pos_hbm, g_hbm, b_hbm, out_hbm, refs):
    B, S, H = in_hbm.shape
    nv = H // _L
    spw = S // _NW
    nk = spw // _C
    pos_v = refs["pos"]
    x_v = refs["x"]
    y_v = refs["y"]
    g_v, b_v = refs["g"], refs["b"]
    mul_v, off_v = refs["mul"], refs["off"]
    semi = refs["semi"]
    semo = refs["semo"]
    wid = lax.axis_index("s") * _NC + lax.axis_index("c")
    base = wid * spw

    pltpu.sync_copy(g_hbm, g_v)
    pltpu.sync_copy(b_hbm, b_v)

    def start_in(k, t):
        s0 = base + k * _C
        pltpu.async_copy(pos_hbm.at[pl.ds(s0, _C)], pos_v[t], semi[t])
        for b in range(B):
            pltpu.async_copy(in_hbm.at[b, pl.ds(s0, _C)], x_v[t][b], semi[t])

    def wait_in(t):
        pltpu.make_async_copy(pos_hbm.at[pl.ds(0, _C)], pos_v[t], semi[t]).wait()
        for b in range(B):
            pltpu.make_async_copy(in_hbm.at[b, pl.ds(0, _C)], x_v[t][b], semi[t]).wait()

    def start_out(k, t):
        s0 = base + k * _C
        for b in range(B):
            pltpu.async_copy(y_v[t][b], out_hbm.at[b, pl.ds(s0, _C)], semo[t])

    def wait_out(t):
        for b in range(B):
            pltpu.make_async_copy(y_v[t][b], out_hbm.at[b, pl.ds(0, _C)], semo[t]).wait()

    def compute(t):
        @plsc.parallel_loop(0, _C, unroll=2)
        def _stats(r):
            acc = [jnp.zeros((_L,), jnp.float32) for _ in range(B)]
            acc2 = [jnp.zeros((_L,), jnp.float32) for _ in range(B)]
            for j in range(nv):
                sl = pl.ds(j * _L, _L)
                p = pos_v[t][r, sl]
                for b in range(B):
                    v = x_v[t][b][r, sl] + p
                    acc[b] = acc[b] + v
                    acc2[b] = acc2[b] + v * v
            for b in range(B):
                mean = _lane_sum(acc[b]) * (1.0 / H)
                var = _lane_sum(acc2[b]) * (1.0 / H) - mean * mean
                rstd = _rsqrt_newton(var + _EPS)
                mul_v[b * _C + r] = rstd
                off_v[b * _C + r] = -mean * rstd

        @plsc.parallel_loop(0, _C, unroll=2)
        def _norm(r):
            a = [mul_v[b * _C + r] for b in range(B)]
            o = [off_v[b * _C + r] for b in range(B)]
            for j in range(nv):
                sl = pl.ds(j * _L, _L)
                p = pos_v[t][r, sl]
                g = g_v[sl]
                bet = b_v[sl]
                for b in range(B):
                    v = x_v[t][b][r, sl] + p
                    y_v[t][b][r, sl] = (v * a[b] + o[b]) * g + bet

    start_in(0, 0)
    start_in(1, 1)

    def ring_body(m, _):
        for t in range(2):
            k = 2 * m + t
            wait_in(t)

            @pl.when(m > 0)
            def _():
                wait_out(t)

            compute(t)
            start_out(k, t)

            @pl.when(k + 2 < nk)
            def _():
                start_in(k + 2, t)

        return 0

    lax.fori_loop(0, nk // 2, ring_body, 0)
    wait_out(0)
    wait_out(1)


def kernel(inputs_embeds, position_table, ln_gamma, ln_beta):
    B, S, H = inputs_embeds.shape
    mesh = plsc.VectorSubcoreMesh(
        core_axis_name="c", subcore_axis_name="s", num_cores=_NC, num_subcores=_NS
    )
    run = functools.partial(
        pl.kernel,
        out_type=jax.ShapeDtypeStruct((B, S, H), jnp.float32),
        mesh=mesh,
        scratch_types=[
            {
                "pos": [pltpu.VMEM((_C, H), jnp.float32) for _ in range(2)],
                "x": [
                    [pltpu.VMEM((_C, H), jnp.float32) for _ in range(B)]
                    for _ in range(2)
                ],
                "y": [
                    [pltpu.VMEM((_C, H), jnp.float32) for _ in range(B)]
                    for _ in range(2)
                ],
                "g": pltpu.VMEM((H,), jnp.float32),
                "b": pltpu.VMEM((H,), jnp.float32),
                "mul": pltpu.VMEM((B * _C, _L), jnp.float32),
                "off": pltpu.VMEM((B * _C, _L), jnp.float32),
                "semi": [pltpu.SemaphoreType.DMA for _ in range(2)],
                "semo": [pltpu.SemaphoreType.DMA for _ in range(2)],
            }
        ],
    )(_sc_body)
    return run(inputs_embeds, position_table[:S], ln_gamma, ln_beta)


# speedup vs baseline: 1.3840x; 1.3840x over previous
import functools

import jax
import jax.numpy as jnp
from jax import lax
from jax.experimental import pallas as pl
from jax.experimental.pallas import tpu as pltpu
from jax.experimental.pallas import tpu_sc as plsc

_EPS = 1e-12
_L = 16
_NC = 2
_NS = 16
_NW = _NC * _NS
_C = 8


def _rsqrt_newton(x):
    xb = lax.bitcast_convert_type(x, jnp.int32)
    y = lax.bitcast_convert_type(
        jnp.int32(0x5F3759DF) - lax.shift_right_arithmetic(xb, 1), jnp.float32
    )
    for _ in range(4):
        y = y * (1.5 - 0.5 * x * y * y)
    return y


def _lane_sum(v):
    ids = lax.iota(jnp.int32, _L)
    dnums = lax.GatherDimensionNumbers(
        offset_dims=(), collapsed_slice_dims=(0,), start_index_map=(0,)
    )
    for k in (1, 2, 4, 8):
        idx = lax.bitwise_xor(ids, jnp.int32(k))
        v = v + lax.gather(
            v,
            idx[:, None],
            dnums,
            slice_sizes=(1,),
            mode=lax.GatherScatterMode.PROMISE_IN_BOUNDS,
        )
    return v


def _sc_body(in_hbm, pos_hbm, g_hbm, b_hbm, out_hbm, refs):
    B, S, H = in_hbm.shape
    nv = H // _L
    spw = S // _NW
    nk = spw // _C
    pos_v = refs["pos"]
    x_v = refs["x"]
    y_v = refs["y"]
    g_v, b_v = refs["g"], refs["b"]
    mul_v, off_v = refs["mul"], refs["off"]
    semi = refs["semi"]
    semo = refs["semo"]
    wid = lax.axis_index("s") * _NC + lax.axis_index("c")
    base = wid * spw

    pltpu.sync_copy(g_hbm, g_v)
    pltpu.sync_copy(b_hbm, b_v)

    def start_in(k, t):
        s0 = base + k * _C
        pltpu.async_copy(pos_hbm.at[pl.ds(s0, _C)], pos_v[t], semi[t])
        for b in range(B):
            pltpu.async_copy(in_hbm.at[b, pl.ds(s0, _C)], x_v[t][b], semi[t])

    def wait_in(t):
        pltpu.make_async_copy(pos_hbm.at[pl.ds(0, _C)], pos_v[t], semi[t]).wait()
        for b in range(B):
            pltpu.make_async_copy(in_hbm.at[b, pl.ds(0, _C)], x_v[t][b], semi[t]).wait()

    def start_out(k, t):
        s0 = base + k * _C
        for b in range(B):
            pltpu.async_copy(y_v[t][b], out_hbm.at[b, pl.ds(s0, _C)], semo[t])

    def wait_out(t):
        for b in range(B):
            pltpu.make_async_copy(y_v[t][b], out_hbm.at[b, pl.ds(0, _C)], semo[t]).wait()

    def compute(t):
        @plsc.parallel_loop(0, _C, unroll=1)
        def _stats(r):
            acc = [jnp.zeros((_L,), jnp.float32) for _ in range(B)]
            acc2 = [jnp.zeros((_L,), jnp.float32) for _ in range(B)]
            for j in range(nv):
                sl = pl.ds(j * _L, _L)
                p = pos_v[t][r, sl]
                for b in range(B):
                    v = x_v[t][b][r, sl] + p
                    y_v[t][b][r, sl] = v
                    acc[b] = acc[b] + v
                    acc2[b] = acc2[b] + v * v
            for b in range(B):
                mean = _lane_sum(acc[b]) * (1.0 / H)
                var = _lane_sum(acc2[b]) * (1.0 / H) - mean * mean
                rstd = _rsqrt_newton(var + _EPS)
                mul_v[b * _C + r] = rstd
                off_v[b * _C + r] = -mean * rstd

        @plsc.parallel_loop(0, _C, unroll=1)
        def _norm(r):
            a = [mul_v[b * _C + r] for b in range(B)]
            o = [off_v[b * _C + r] for b in range(B)]
            for j in range(nv):
                sl = pl.ds(j * _L, _L)
                g = g_v[sl]
                bet = b_v[sl]
                for b in range(B):
                    v = y_v[t][b][r, sl]
                    y_v[t][b][r, sl] = (v * a[b] + o[b]) * g + bet

    start_in(0, 0)
    start_in(1, 1)

    def ring_body(m, _):
        for t in range(2):
            k = 2 * m + t
            wait_in(t)

            @pl.when(m > 0)
            def _():
                wait_out(t)

            compute(t)
            start_out(k, t)

            @pl.when(k + 2 < nk)
            def _():
                start_in(k + 2, t)

        return 0

    lax.fori_loop(0, nk // 2, ring_body, 0)
    wait_out(0)
    wait_out(1)


def kernel(inputs_embeds, position_table, ln_gamma, ln_beta):
    B, S, H = inputs_embeds.shape
    mesh = plsc.VectorSubcoreMesh(
        core_axis_name="c", subcore_axis_name="s", num_cores=_NC, num_subcores=_NS
    )
    run = functools.partial(
        pl.kernel,
        out_type=jax.ShapeDtypeStruct((B, S, H), jnp.float32),
        mesh=mesh,
        scratch_types=[
            {
                "pos": [pltpu.VMEM((_C, H), jnp.float32) for _ in range(2)],
                "x": [
                    [pltpu.VMEM((_C, H), jnp.float32) for _ in range(B)]
                    for _ in range(2)
                ],
                "y": [
                    [pltpu.VMEM((_C, H), jnp.float32) for _ in range(B)]
                    for _ in range(2)
                ],
                "g": pltpu.VMEM((H,), jnp.float32),
                "b": pltpu.VMEM((H,), jnp.float32),
                "mul": pltpu.VMEM((B * _C, _L), jnp.float32),
                "off": pltpu.VMEM((B * _C, _L), jnp.float32),
                "semi": [pltpu.SemaphoreType.DMA for _ in range(2)],
                "semo": [pltpu.SemaphoreType.DMA for _ in range(2)],
            }
        ],
    )(_sc_body)
    return run(inputs_embeds, position_table[:S], ln_gamma, ln_beta)


# speedup vs baseline: 1.4603x; 1.0551x over previous
import functools

import jax
import jax.numpy as jnp
from jax import lax
from jax.experimental import pallas as pl
from jax.experimental.pallas import tpu as pltpu
from jax.experimental.pallas import tpu_sc as plsc

_EPS = 1e-12
_L = 16
_NC = 2
_NS = 16
_NW = _NC * _NS
_C = 8


def _rsqrt_newton(x):
    xb = lax.bitcast_convert_type(x, jnp.int32)
    y = lax.bitcast_convert_type(
        jnp.int32(0x5F3759DF) - lax.shift_right_arithmetic(xb, 1), jnp.float32
    )
    for _ in range(4):
        y = y * (1.5 - 0.5 * x * y * y)
    return y


def _lane_sum(v):
    ids = lax.iota(jnp.int32, _L)
    dnums = lax.GatherDimensionNumbers(
        offset_dims=(), collapsed_slice_dims=(0,), start_index_map=(0,)
    )
    for k in (1, 2, 4, 8):
        idx = lax.bitwise_xor(ids, jnp.int32(k))
        v = v + lax.gather(
            v,
            idx[:, None],
            dnums,
            slice_sizes=(1,),
            mode=lax.GatherScatterMode.PROMISE_IN_BOUNDS,
        )
    return v


def _sc_body(in_hbm, pos_hbm, g_hbm, b_hbm, out_hbm, refs):
    B, S, H = in_hbm.shape
    nv = H // _L
    spw = S // _NW
    nk = spw // _C
    pos_v = refs["pos"]
    x_v = refs["x"]
    y_v = refs["y"]
    g_v, b_v = refs["g"], refs["b"]
    mul_v, off_v = refs["mul"], refs["off"]
    semi = refs["semi"]
    semo = refs["semo"]
    wid = lax.axis_index("s") * _NC + lax.axis_index("c")
    base = wid * spw

    pltpu.sync_copy(g_hbm, g_v)
    pltpu.sync_copy(b_hbm, b_v)

    def start_in(k, t):
        s0 = base + k * _C
        pltpu.async_copy(pos_hbm.at[pl.ds(s0, _C)], pos_v[t], semi[t])
        pltpu.async_copy(in_hbm.at[:, pl.ds(s0, _C)], x_v[t], semi[t])

    def wait_in(t):
        pltpu.make_async_copy(pos_hbm.at[pl.ds(0, _C)], pos_v[t], semi[t]).wait()
        pltpu.make_async_copy(in_hbm.at[:, pl.ds(0, _C)], x_v[t], semi[t]).wait()

    def start_out(k, t):
        s0 = base + k * _C
        pltpu.async_copy(y_v[t], out_hbm.at[:, pl.ds(s0, _C)], semo[t])

    def wait_out(t):
        pltpu.make_async_copy(y_v[t], out_hbm.at[:, pl.ds(0, _C)], semo[t]).wait()

    def compute(t):
        @plsc.parallel_loop(0, _C, unroll=1)
        def _stats(r):
            acc = [jnp.zeros((_L,), jnp.float32) for _ in range(B)]
            acc2 = [jnp.zeros((_L,), jnp.float32) for _ in range(B)]
            for j in range(nv):
                sl = pl.ds(j * _L, _L)
                p = pos_v[t][r, sl]
                for b in range(B):
                    v = x_v[t][b, r, sl] + p
                    acc[b] = acc[b] + v
                    acc2[b] = acc2[b] + v * v
            for b in range(B):
                mean = _lane_sum(acc[b]) * (1.0 / H)
                var = _lane_sum(acc2[b]) * (1.0 / H) - mean * mean
                rstd = _rsqrt_newton(var + _EPS)
                mul_v[b * _C + r] = rstd
                off_v[b * _C + r] = -mean * rstd

        @plsc.parallel_loop(0, _C, unroll=1)
        def _norm(r):
            a = [mul_v[b * _C + r] for b in range(B)]
            o = [off_v[b * _C + r] for b in range(B)]
            for j in range(nv):
                sl = pl.ds(j * _L, _L)
                p = pos_v[t][r, sl]
                g = g_v[sl]
                bet = b_v[sl]
                for b in range(B):
                    v = x_v[t][b, r, sl] + p
                    y_v[t][b, r, sl] = (v * a[b] + o[b]) * g + bet

    start_in(0, 0)
    start_in(1, 1)

    def ring_body(m, _):
        for t in range(2):
            k = 2 * m + t
            wait_in(t)

            @pl.when(m > 0)
            def _():
                wait_out(t)

            compute(t)
            start_out(k, t)

            @pl.when(k + 2 < nk)
            def _():
                start_in(k + 2, t)

        return 0

    lax.fori_loop(0, nk // 2, ring_body, 0)
    wait_out(0)
    wait_out(1)


def kernel(inputs_embeds, position_table, ln_gamma, ln_beta):
    B, S, H = inputs_embeds.shape
    mesh = plsc.VectorSubcoreMesh(
        core_axis_name="c", subcore_axis_name="s", num_cores=_NC, num_subcores=_NS
    )
    run = functools.partial(
        pl.kernel,
        out_type=jax.ShapeDtypeStruct((B, S, H), jnp.float32),
        mesh=mesh,
        scratch_types=[
            {
                "pos": [pltpu.VMEM((_C, H), jnp.float32) for _ in range(2)],
                "x": [pltpu.VMEM((B, _C, H), jnp.float32) for _ in range(2)],
                "y": [pltpu.VMEM((B, _C, H), jnp.float32) for _ in range(2)],
                "g": pltpu.VMEM((H,), jnp.float32),
                "b": pltpu.VMEM((H,), jnp.float32),
                "mul": pltpu.VMEM((B * _C, _L), jnp.float32),
                "off": pltpu.VMEM((B * _C, _L), jnp.float32),
                "semi": [pltpu.SemaphoreType.DMA for _ in range(2)],
                "semo": [pltpu.SemaphoreType.DMA for _ in range(2)],
            }
        ],
    )(_sc_body)
    return run(inputs_embeds, position_table[:S], ln_gamma, ln_beta)


# speedup vs baseline: 1.4728x; 1.0085x over previous
import functools

import jax
import jax.numpy as jnp
from jax import lax
from jax.experimental import pallas as pl
from jax.experimental.pallas import tpu as pltpu
from jax.experimental.pallas import tpu_sc as plsc

_EPS = 1e-12
_L = 16
_NC = 2
_NS = 16
_NW = _NC * _NS
_C = 8


def _rsqrt_newton(x):
    xb = lax.bitcast_convert_type(x, jnp.int32)
    y = lax.bitcast_convert_type(
        jnp.int32(0x5F3759DF) - lax.shift_right_arithmetic(xb, 1), jnp.float32
    )
    for _ in range(4):
        y = y * (1.5 - 0.5 * x * y * y)
    return y


def _lane_sum(v):
    ids = lax.iota(jnp.int32, _L)
    dnums = lax.GatherDimensionNumbers(
        offset_dims=(), collapsed_slice_dims=(0,), start_index_map=(0,)
    )
    for k in (1, 2, 4, 8):
        idx = lax.bitwise_xor(ids, jnp.int32(k))
        v = v + lax.gather(
            v,
            idx[:, None],
            dnums,
            slice_sizes=(1,),
            mode=lax.GatherScatterMode.PROMISE_IN_BOUNDS,
        )
    return v


def _sc_body(in_hbm, pos_hbm, g_hbm, b_hbm, out_hbm, refs):
    B, S, H = in_hbm.shape
    nv = H // _L
    spw = S // _NW
    nk = spw // _C
    pos_v = refs["pos"]
    x_v = refs["x"]
    y_v = refs["y"]
    g_v, b_v = refs["g"], refs["b"]
    mul_v, off_v = refs["mul"], refs["off"]
    semi = refs["semi"]
    semo = refs["semo"]
    wid = lax.axis_index("s") * _NC + lax.axis_index("c")
    base = wid * spw

    pltpu.sync_copy(g_hbm, g_v)
    pltpu.sync_copy(b_hbm, b_v)

    def start_in(k, t):
        s0 = base + k * _C
        pltpu.async_copy(pos_hbm.at[pl.ds(s0, _C)], pos_v[t], semi[t])
        pltpu.async_copy(in_hbm.at[:, pl.ds(s0, _C)], x_v[t], semi[t])

    def wait_in(t):
        pltpu.make_async_copy(pos_hbm.at[pl.ds(0, _C)], pos_v[t], semi[t]).wait()
        pltpu.make_async_copy(in_hbm.at[:, pl.ds(0, _C)], x_v[t], semi[t]).wait()

    def start_out(k, t):
        s0 = base + k * _C
        pltpu.async_copy(y_v[t], out_hbm.at[:, pl.ds(s0, _C)], semo[t])

    def wait_out(t):
        pltpu.make_async_copy(y_v[t], out_hbm.at[:, pl.ds(0, _C)], semo[t]).wait()

    def compute(t):
        @plsc.parallel_loop(0, _C, unroll=1)
        def _rows(r):
            acc = [jnp.zeros((_L,), jnp.float32) for _ in range(B)]
            acc2 = [jnp.zeros((_L,), jnp.float32) for _ in range(B)]
            for j in range(nv):
                sl = pl.ds(j * _L, _L)
                p = pos_v[t][r, sl]
                for b in range(B):
                    v = x_v[t][b, r, sl] + p
                    acc[b] = acc[b] + v
                    acc2[b] = acc2[b] + v * v
            a = []
            o = []
            for b in range(B):
                mean = _lane_sum(acc[b]) * (1.0 / H)
                var = _lane_sum(acc2[b]) * (1.0 / H) - mean * mean
                rstd = _rsqrt_newton(var + _EPS)
                a.append(rstd)
                o.append(-mean * rstd)
            for j in range(nv):
                sl = pl.ds(j * _L, _L)
                p = pos_v[t][r, sl]
                g = g_v[sl]
                bet = b_v[sl]
                for b in range(B):
                    v = x_v[t][b, r, sl] + p
                    y_v[t][b, r, sl] = (v * a[b] + o[b]) * g + bet

    start_in(0, 0)
    start_in(1, 1)

    def ring_body(m, _):
        for t in range(2):
            k = 2 * m + t
            wait_in(t)

            @pl.when(m > 0)
            def _():
                wait_out(t)

            compute(t)
            start_out(k, t)

            @pl.when(k + 2 < nk)
            def _():
                start_in(k + 2, t)

        return 0

    lax.fori_loop(0, nk // 2, ring_body, 0)
    wait_out(0)
    wait_out(1)


def kernel(inputs_embeds, position_table, ln_gamma, ln_beta):
    B, S, H = inputs_embeds.shape
    mesh = plsc.VectorSubcoreMesh(
        core_axis_name="c", subcore_axis_name="s", num_cores=_NC, num_subcores=_NS
    )
    run = functools.partial(
        pl.kernel,
        out_type=jax.ShapeDtypeStruct((B, S, H), jnp.float32),
        mesh=mesh,
        scratch_types=[
            {
                "pos": [pltpu.VMEM((_C, H), jnp.float32) for _ in range(2)],
                "x": [pltpu.VMEM((B, _C, H), jnp.float32) for _ in range(2)],
                "y": [pltpu.VMEM((B, _C, H), jnp.float32) for _ in range(2)],
                "g": pltpu.VMEM((H,), jnp.float32),
                "b": pltpu.VMEM((H,), jnp.float32),
                "mul": pltpu.VMEM((B * _C, _L), jnp.float32),
                "off": pltpu.VMEM((B * _C, _L), jnp.float32),
                "semi": [pltpu.SemaphoreType.DMA for _ in range(2)],
                "semo": [pltpu.SemaphoreType.DMA for _ in range(2)],
            }
        ],
    )(_sc_body)
    return run(inputs_embeds, position_table[:S], ln_gamma, ln_beta)


# speedup vs baseline: 1.5209x; 1.0327x over previous
import functools

import jax
import jax.numpy as jnp
from jax import lax
from jax.experimental import pallas as pl
from jax.experimental.pallas import tpu as pltpu
from jax.experimental.pallas import tpu_sc as plsc

_EPS = 1e-12
_L = 16
_NC = 2
_NS = 16
_NW = _NC * _NS
_C = 8


def _rsqrt_newton(x):
    xb = lax.bitcast_convert_type(x, jnp.int32)
    y = lax.bitcast_convert_type(
        jnp.int32(0x5F3759DF) - lax.shift_right_arithmetic(xb, 1), jnp.float32
    )
    for _ in range(4):
        y = y * (1.5 - 0.5 * x * y * y)
    return y


def _lane_sum(v):
    ids = lax.iota(jnp.int32, _L)
    dnums = lax.GatherDimensionNumbers(
        offset_dims=(), collapsed_slice_dims=(0,), start_index_map=(0,)
    )
    for k in (1, 2, 4, 8):
        idx = lax.bitwise_xor(ids, jnp.int32(k))
        v = v + lax.gather(
            v,
            idx[:, None],
            dnums,
            slice_sizes=(1,),
            mode=lax.GatherScatterMode.PROMISE_IN_BOUNDS,
        )
    return v


def _sc_body(in_hbm, pos_hbm, g_hbm, b_hbm, out_hbm, refs):
    B, S, H = in_hbm.shape
    nv = H // _L
    spw = S // _NW
    nk = spw // _C
    pos_v = refs["pos"]
    x_v = refs["x"]
    y_v = refs["y"]
    g_v, b_v = refs["g"], refs["b"]
    semi = refs["semi"]
    semo = refs["semo"]
    wid = lax.axis_index("s") * _NC + lax.axis_index("c")
    base = wid * spw

    pltpu.sync_copy(g_hbm, g_v)
    pltpu.sync_copy(b_hbm, b_v)

    def start_in(k, t):
        s0 = base + k * _C
        pltpu.async_copy(pos_hbm.at[pl.ds(s0, _C)], pos_v[t], semi[t])
        pltpu.async_copy(in_hbm.at[:, pl.ds(s0, _C)], x_v[t], semi[t])

    def wait_in(t):
        pltpu.make_async_copy(pos_hbm.at[pl.ds(0, _C)], pos_v[t], semi[t]).wait()
        pltpu.make_async_copy(in_hbm.at[:, pl.ds(0, _C)], x_v[t], semi[t]).wait()

    def start_out(k, t):
        s0 = base + k * _C
        pltpu.async_copy(y_v[t], out_hbm.at[:, pl.ds(s0, _C)], semo[t])

    def wait_out(t):
        pltpu.make_async_copy(y_v[t], out_hbm.at[:, pl.ds(0, _C)], semo[t]).wait()

    def compute(t):
        @plsc.parallel_loop(0, _C, unroll=1)
        def _rows(r):
            acc = [jnp.zeros((_L,), jnp.float32) for _ in range(B)]
            acc2 = [jnp.zeros((_L,), jnp.float32) for _ in range(B)]
            for j in range(nv):
                sl = pl.ds(j * _L, _L)
                p = pos_v[t][r, sl]
                for b in range(B):
                    v = x_v[t][b, r, sl] + p
                    acc[b] = acc[b] + v
                    acc2[b] = acc2[b] + v * v
            a = []
            o = []
            for b in range(B):
                mean = _lane_sum(acc[b]) * (1.0 / H)
                var = _lane_sum(acc2[b]) * (1.0 / H) - mean * mean
                rstd = _rsqrt_newton(var + _EPS)
                a.append(rstd)
                o.append(-mean * rstd)
            for j in range(nv):
                sl = pl.ds(j * _L, _L)
                p = pos_v[t][r, sl]
                g = g_v[sl]
                bet = b_v[sl]
                for b in range(B):
                    v = x_v[t][b, r, sl] + p
                    y_v[t][b, r, sl] = (v * a[b] + o[b]) * g + bet

    start_in(0, 0)
    start_in(1, 1)

    def ring_body(m, _):
        for t in range(2):
            k = 2 * m + t
            wait_in(t)

            @pl.when(m > 0)
            def _():
                wait_out(t)

            compute(t)
            start_out(k, t)

            @pl.when(k + 2 < nk)
            def _():
                start_in(k + 2, t)

        return 0

    lax.fori_loop(0, nk // 2, ring_body, 0)
    wait_out(0)
    wait_out(1)


def kernel(inputs_embeds, position_table, ln_gamma, ln_beta):
    B, S, H = inputs_embeds.shape
    mesh = plsc.VectorSubcoreMesh(
        core_axis_name="c", subcore_axis_name="s", num_cores=_NC, num_subcores=_NS
    )
    run = functools.partial(
        pl.kernel,
        out_type=jax.ShapeDtypeStruct((B, S, H), jnp.float32),
        mesh=mesh,
        scratch_types=[
            {
                "pos": [pltpu.VMEM((_C, H), jnp.float32) for _ in range(2)],
                "x": [pltpu.VMEM((B, _C, H), jnp.float32) for _ in range(2)],
                "y": [pltpu.VMEM((B, _C, H), jnp.float32) for _ in range(2)],
                "g": pltpu.VMEM((H,), jnp.float32),
                "b": pltpu.VMEM((H,), jnp.float32),
                "semi": [pltpu.SemaphoreType.DMA for _ in range(2)],
                "semo": [pltpu.SemaphoreType.DMA for _ in range(2)],
            }
        ],
    )(_sc_body)
    return run(inputs_embeds, position_table[:S], ln_gamma, ln_beta)
